# Initial kernel scaffold; baseline (speedup 1.0000x reference)
import jax
import jax.numpy as jnp
from jax.experimental import pallas as pl


def kernel(x, edge_index, W1, W2, W3):
    raise NotImplementedError("write your pallas kernel here")


# speedup vs baseline: 5.9826x; 5.9826x over previous
import functools

import jax
import jax.numpy as jnp
from jax import lax
from jax.experimental import pallas as pl
from jax.experimental.pallas import tpu as pltpu
from jax.experimental.pallas import tpu_sc as plsc

N_NODES = 10000
N_PAD = 10240
D = 256
DH = 128
E_CHUNK = 128
E_PAD = 172032
N_TILES = 16
ROWS_PER_TILE = N_PAD // N_TILES
AGG_CHUNKS = E_PAD // (N_TILES * E_CHUNK)
DEG_CHUNKS = E_PAD // (2 * N_TILES * E_CHUNK)
DUMMY_ROW = N_NODES + 1
BM = 256
GRID = N_PAD // BM


@functools.cache
def _mesh():
    return plsc.VectorSubcoreMesh(
        core_axis_name="c", subcore_axis_name="s", num_cores=2, num_subcores=16)


def _deg_body(rowp, ones_hbm, zeros_hbm, deg_out, acc, ones_v, rowbuf):
    c = lax.axis_index("c")
    s = lax.axis_index("s")
    r0 = s * ROWS_PER_TILE
    pltpu.sync_copy(zeros_hbm, acc.at[pl.ds(r0, ROWS_PER_TILE)])
    pltpu.sync_copy(ones_hbm, ones_v)
    plsc.subcore_barrier()
    e0 = (c * N_TILES + s) * (DEG_CHUNKS * E_CHUNK)

    def step(j, carry):
        base = e0 + j * E_CHUNK
        pltpu.sync_copy(rowp.at[pl.ds(base, E_CHUNK)], rowbuf)
        pltpu.sync_copy(ones_v, acc.at[rowbuf], add=True)
        return carry

    lax.fori_loop(0, DEG_CHUNKS, step, 0)
    plsc.subcore_barrier()
    pltpu.sync_copy(acc.at[pl.ds(r0, ROWS_PER_TILE)],
                    deg_out.at[c, pl.ds(r0, ROWS_PER_TILE)])


def _deg_call(rowp, ones_dh, zeros_dh):
    f = pl.kernel(
        _deg_body,
        out_type=jax.ShapeDtypeStruct((2, N_PAD, DH), jnp.float32),
        mesh=_mesh(),
        scratch_types=[
            pltpu.VMEM_SHARED((N_PAD, DH), jnp.float32),
            pltpu.VMEM((E_CHUNK, DH), jnp.float32),
            pltpu.VMEM((E_CHUNK,), jnp.int32),
        ],
    )
    return f(rowp, ones_dh, zeros_dh)


def _agg_body(sup2, col2, rowp, zeros_hbm, agg_out,
              acc, rows_v, colbuf, rowbuf, sem):
    c = lax.axis_index("c")
    s = lax.axis_index("s")
    r0 = s * ROWS_PER_TILE
    pltpu.sync_copy(zeros_hbm, acc.at[pl.ds(r0, ROWS_PER_TILE)])
    plsc.subcore_barrier()
    e0 = s * (AGG_CHUNKS * E_CHUNK)

    def step(j, carry):
        base = e0 + j * E_CHUNK
        pltpu.sync_copy(col2.at[c, pl.ds(base, E_CHUNK)], colbuf)
        pltpu.sync_copy(rowp.at[pl.ds(base, E_CHUNK)], rowbuf)
        pltpu.async_copy(sup2.at[colbuf], rows_v, sem).wait()
        pltpu.sync_copy(rows_v, acc.at[rowbuf], add=True)
        return carry

    lax.fori_loop(0, AGG_CHUNKS, step, 0)
    plsc.subcore_barrier()
    pltpu.sync_copy(acc.at[pl.ds(r0, ROWS_PER_TILE)],
                    agg_out.at[c, pl.ds(r0, ROWS_PER_TILE)])


def _agg_call(sup2, col2, rowp, zeros_dh):
    f = pl.kernel(
        _agg_body,
        out_type=jax.ShapeDtypeStruct((2, N_PAD, DH), jnp.float32),
        mesh=_mesh(),
        scratch_types=[
            pltpu.VMEM_SHARED((N_PAD, DH), jnp.float32),
            pltpu.VMEM((E_CHUNK, DH), jnp.float32),
            pltpu.VMEM((E_CHUNK,), jnp.int32),
            pltpu.VMEM((E_CHUNK,), jnp.int32),
            pltpu.SemaphoreType.DMA,
        ],
    )
    return f(sup2, col2, rowp, zeros_dh)


def _dinv_block(deg_ref):
    deg = deg_ref[0, :, 0:1] + deg_ref[1, :, 0:1]
    return jnp.where(deg > 0.0, lax.rsqrt(jnp.maximum(deg, 1.0)), 0.0)


def _mm1_body(x_ref, w_ref, deg_ref, sup_ref):
    dinv = _dinv_block(deg_ref)
    sup = jnp.dot(x_ref[...], w_ref[...], preferred_element_type=jnp.float32)
    sup = sup * dinv
    sup_ref[0] = sup[:, :DH]
    sup_ref[1] = sup[:, DH:]


def _mm2_body(x_ref, agg_ref, deg_ref, w_ref, xn_ref, sup_ref):
    dinv = _dinv_block(deg_ref)
    agg = jnp.concatenate([agg_ref[0], agg_ref[1]], axis=1)
    xn = x_ref[...] + jnp.maximum(agg * dinv, 0.0)
    xn_ref[...] = xn
    sup = jnp.dot(xn, w_ref[...], preferred_element_type=jnp.float32)
    sup = sup * dinv
    sup_ref[0] = sup[:, :DH]
    sup_ref[1] = sup[:, DH:]


def _fin_body(agg_ref, deg_ref, out_ref):
    dinv = _dinv_block(deg_ref)
    out_ref[...] = jnp.concatenate([agg_ref[0], agg_ref[1]], axis=1) * dinv


_row_spec = pl.BlockSpec((BM, D), lambda i: (i, 0))
_half2_spec = pl.BlockSpec((2, BM, DH), lambda i: (0, i, 0))
_deg_spec = pl.BlockSpec((2, BM, DH), lambda i: (0, i, 0))
_w_spec = pl.BlockSpec((D, D), lambda i: (0, 0))

_half2_out = jax.ShapeDtypeStruct((2, N_PAD, DH), jnp.float32)
_row_out = jax.ShapeDtypeStruct((N_PAD, D), jnp.float32)

_mm1 = pl.pallas_call(
    _mm1_body,
    grid=(GRID,),
    in_specs=[_row_spec, _w_spec, _deg_spec],
    out_specs=_half2_spec,
    out_shape=_half2_out,
)

_mm2 = pl.pallas_call(
    _mm2_body,
    grid=(GRID,),
    in_specs=[_row_spec, _half2_spec, _deg_spec, _w_spec],
    out_specs=[_row_spec, _half2_spec],
    out_shape=[_row_out, _half2_out],
)

_fin = pl.pallas_call(
    _fin_body,
    grid=(GRID,),
    in_specs=[_half2_spec, _deg_spec],
    out_specs=_row_spec,
    out_shape=_row_out,
)


def kernel(x, edge_index, W1, W2, W3):
    x = x.astype(jnp.float32)
    xp = jnp.pad(x, ((0, N_PAD - N_NODES), (0, 0)))
    loop = jnp.arange(N_NODES, dtype=jnp.int32)
    n_fill = E_PAD - (edge_index.shape[1] + N_NODES)
    rowp = jnp.concatenate(
        [edge_index[0], loop, jnp.full((n_fill,), DUMMY_ROW, jnp.int32)])
    colp = jnp.concatenate(
        [edge_index[1], loop, jnp.zeros((n_fill,), jnp.int32)])
    col2 = jnp.stack([colp, colp + N_PAD])
    ones_dh = jnp.ones((E_CHUNK, DH), jnp.float32)
    zeros_dh = jnp.zeros((ROWS_PER_TILE, DH), jnp.float32)

    deg = _deg_call(rowp, ones_dh, zeros_dh)
    sup = _mm1(xp, W1, deg).reshape(2 * N_PAD, DH)
    agg = _agg_call(sup, col2, rowp, zeros_dh)
    x1, sup = _mm2(xp, agg, deg, W2)
    agg = _agg_call(sup.reshape(2 * N_PAD, DH), col2, rowp, zeros_dh)
    _, sup = _mm2(x1, agg, deg, W3)
    agg = _agg_call(sup.reshape(2 * N_PAD, DH), col2, rowp, zeros_dh)
    logits = _fin(agg, deg)
    return logits[:N_NODES]


# speedup vs baseline: 6.8687x; 1.1481x over previous
import functools

import jax
import jax.numpy as jnp
from jax import lax
from jax.experimental import pallas as pl
from jax.experimental.pallas import tpu as pltpu
from jax.experimental.pallas import tpu_sc as plsc

N_NODES = 10000
N_PAD = 10240
D = 256
DH = 128
E_CHUNK = 128
E_PAD = 172032
N_TILES = 16
ROWS_PER_TILE = N_PAD // N_TILES
AGG_CHUNKS = E_PAD // (N_TILES * E_CHUNK)
DEG_CHUNKS = E_PAD // (2 * N_TILES * E_CHUNK)
DUMMY_ROW = N_NODES + 1
BM = 256
GRID = N_PAD // BM


@functools.cache
def _mesh():
    return plsc.VectorSubcoreMesh(
        core_axis_name="c", subcore_axis_name="s", num_cores=2, num_subcores=16)


def _deg_body(rowp, ones_hbm, zeros_hbm, deg_out, acc, ones_v, rowbuf):
    c = lax.axis_index("c")
    s = lax.axis_index("s")
    r0 = s * ROWS_PER_TILE
    pltpu.sync_copy(zeros_hbm, acc.at[pl.ds(r0, ROWS_PER_TILE)])
    pltpu.sync_copy(ones_hbm, ones_v)
    plsc.subcore_barrier()
    e0 = (c * N_TILES + s) * (DEG_CHUNKS * E_CHUNK)

    def step(j, carry):
        base = e0 + j * E_CHUNK
        pltpu.sync_copy(rowp.at[pl.ds(base, E_CHUNK)], rowbuf)
        pltpu.sync_copy(ones_v, acc.at[rowbuf], add=True)
        return carry

    lax.fori_loop(0, DEG_CHUNKS, step, 0)
    plsc.subcore_barrier()
    pltpu.sync_copy(acc.at[pl.ds(r0, ROWS_PER_TILE)],
                    deg_out.at[c, pl.ds(r0, ROWS_PER_TILE)])


def _deg_call(rowp, ones_dh, zeros_dh):
    f = pl.kernel(
        _deg_body,
        out_type=jax.ShapeDtypeStruct((2, N_PAD, DH), jnp.float32),
        mesh=_mesh(),
        scratch_types=[
            pltpu.VMEM_SHARED((N_PAD, DH), jnp.float32),
            pltpu.VMEM((E_CHUNK, DH), jnp.float32),
            pltpu.VMEM((E_CHUNK,), jnp.int32),
        ],
    )
    return f(rowp, ones_dh, zeros_dh)


G = 2
N_GROUPS = AGG_CHUNKS // G


def _agg_body(sup2, col3, row3, zeros_hbm, agg_out,
              acc, rows_v, colbuf, rowbuf, sem0, sem1):
    c = lax.axis_index("c")
    s = lax.axis_index("s")
    r0 = s * ROWS_PER_TILE
    pltpu.sync_copy(zeros_hbm, acc.at[pl.ds(r0, ROWS_PER_TILE)])
    plsc.subcore_barrier()
    sems = (sem0, sem1)

    def group(g, carry):
        gchunk = s * AGG_CHUNKS + g * G
        pltpu.sync_copy(col3.at[c, pl.ds(gchunk, G)], colbuf)
        pltpu.sync_copy(row3.at[pl.ds(gchunk, G)], rowbuf)
        descs = [
            pltpu.async_copy(sup2.at[colbuf.at[k]], rows_v.at[k], sems[k])
            for k in range(G)
        ]
        for k in range(G):
            descs[k].wait()
            pltpu.sync_copy(rows_v.at[k], acc.at[rowbuf.at[k]], add=True)
        return carry

    lax.fori_loop(0, N_GROUPS, group, 0)
    plsc.subcore_barrier()
    pltpu.sync_copy(acc.at[pl.ds(r0, ROWS_PER_TILE)],
                    agg_out.at[c, pl.ds(r0, ROWS_PER_TILE)])


def _agg_call(sup2, col3, row3, zeros_dh):
    f = pl.kernel(
        _agg_body,
        out_type=jax.ShapeDtypeStruct((2, N_PAD, DH), jnp.float32),
        mesh=_mesh(),
        scratch_types=[
            pltpu.VMEM_SHARED((N_PAD, DH), jnp.float32),
            pltpu.VMEM((G, E_CHUNK, DH), jnp.float32),
            pltpu.VMEM((G, E_CHUNK), jnp.int32),
            pltpu.VMEM((G, E_CHUNK), jnp.int32),
            pltpu.SemaphoreType.DMA,
            pltpu.SemaphoreType.DMA,
        ],
    )
    return f(sup2, col3, row3, zeros_dh)


def _dinv_block(deg_ref):
    deg = deg_ref[0, :, 0:1] + deg_ref[1, :, 0:1]
    return jnp.where(deg > 0.0, lax.rsqrt(jnp.maximum(deg, 1.0)), 0.0)


def _mm1_body(x_ref, w_ref, deg_ref, sup_ref):
    dinv = _dinv_block(deg_ref)
    sup = jnp.dot(x_ref[...], w_ref[...], preferred_element_type=jnp.float32)
    sup = sup * dinv
    sup_ref[0] = sup[:, :DH]
    sup_ref[1] = sup[:, DH:]


def _mm2_body(x_ref, agg_ref, deg_ref, w_ref, xn_ref, sup_ref):
    dinv = _dinv_block(deg_ref)
    agg = jnp.concatenate([agg_ref[0], agg_ref[1]], axis=1)
    xn = x_ref[...] + jnp.maximum(agg * dinv, 0.0)
    xn_ref[...] = xn
    sup = jnp.dot(xn, w_ref[...], preferred_element_type=jnp.float32)
    sup = sup * dinv
    sup_ref[0] = sup[:, :DH]
    sup_ref[1] = sup[:, DH:]


def _fin_body(agg_ref, deg_ref, out_ref):
    dinv = _dinv_block(deg_ref)
    out_ref[...] = jnp.concatenate([agg_ref[0], agg_ref[1]], axis=1) * dinv


_row_spec = pl.BlockSpec((BM, D), lambda i: (i, 0))
_half2_spec = pl.BlockSpec((2, BM, DH), lambda i: (0, i, 0))
_deg_spec = pl.BlockSpec((2, BM, DH), lambda i: (0, i, 0))
_w_spec = pl.BlockSpec((D, D), lambda i: (0, 0))

_half2_out = jax.ShapeDtypeStruct((2, N_PAD, DH), jnp.float32)
_row_out = jax.ShapeDtypeStruct((N_PAD, D), jnp.float32)

_mm1 = pl.pallas_call(
    _mm1_body,
    grid=(GRID,),
    in_specs=[_row_spec, _w_spec, _deg_spec],
    out_specs=_half2_spec,
    out_shape=_half2_out,
)

_mm2 = pl.pallas_call(
    _mm2_body,
    grid=(GRID,),
    in_specs=[_row_spec, _half2_spec, _deg_spec, _w_spec],
    out_specs=[_row_spec, _half2_spec],
    out_shape=[_row_out, _half2_out],
)

_fin = pl.pallas_call(
    _fin_body,
    grid=(GRID,),
    in_specs=[_half2_spec, _deg_spec],
    out_specs=_row_spec,
    out_shape=_row_out,
)


def kernel(x, edge_index, W1, W2, W3):
    x = x.astype(jnp.float32)
    xp = jnp.pad(x, ((0, N_PAD - N_NODES), (0, 0)))
    loop = jnp.arange(N_NODES, dtype=jnp.int32)
    n_fill = E_PAD - (edge_index.shape[1] + N_NODES)
    rowp = jnp.concatenate(
        [edge_index[0], loop, jnp.full((n_fill,), DUMMY_ROW, jnp.int32)])
    colp = jnp.concatenate(
        [edge_index[1], loop, jnp.zeros((n_fill,), jnp.int32)])
    col3 = jnp.stack([colp, colp + N_PAD]).reshape(2, E_PAD // E_CHUNK, E_CHUNK)
    row3 = rowp.reshape(E_PAD // E_CHUNK, E_CHUNK)
    ones_dh = jnp.ones((E_CHUNK, DH), jnp.float32)
    zeros_dh = jnp.zeros((ROWS_PER_TILE, DH), jnp.float32)

    deg = _deg_call(rowp, ones_dh, zeros_dh)
    sup = _mm1(xp, W1, deg).reshape(2 * N_PAD, DH)
    agg = _agg_call(sup, col3, row3, zeros_dh)
    x1, sup = _mm2(xp, agg, deg, W2)
    agg = _agg_call(sup.reshape(2 * N_PAD, DH), col3, row3, zeros_dh)
    _, sup = _mm2(x1, agg, deg, W3)
    agg = _agg_call(sup.reshape(2 * N_PAD, DH), col3, row3, zeros_dh)
    logits = _fin(agg, deg)
    return logits[:N_NODES]


# speedup vs baseline: 7.5773x; 1.1032x over previous
import functools

import jax
import jax.numpy as jnp
from jax import lax
from jax.experimental import pallas as pl
from jax.experimental.pallas import tpu as pltpu
from jax.experimental.pallas import tpu_sc as plsc

N_NODES = 10000
N_PAD = 10240
D = 256
DH = 128
E_CHUNK = 128
E_PAD = 172032
N_TILES = 16
ROWS_PER_TILE = N_PAD // N_TILES
AGG_CHUNKS = E_PAD // (N_TILES * E_CHUNK)
DEG_CHUNKS = E_PAD // (2 * N_TILES * E_CHUNK)
DUMMY_ROW = N_NODES + 1
BM = 256
GRID = N_PAD // BM


@functools.cache
def _mesh():
    return plsc.VectorSubcoreMesh(
        core_axis_name="c", subcore_axis_name="s", num_cores=2, num_subcores=16)


def _deg_body(rowp, ones_hbm, zeros_hbm, deg_out, acc, ones_v, rowbuf):
    c = lax.axis_index("c")
    s = lax.axis_index("s")
    r0 = s * ROWS_PER_TILE
    pltpu.sync_copy(zeros_hbm, acc.at[pl.ds(r0, ROWS_PER_TILE)])
    pltpu.sync_copy(ones_hbm, ones_v)
    plsc.subcore_barrier()
    e0 = (c * N_TILES + s) * (DEG_CHUNKS * E_CHUNK)

    def step(j, carry):
        base = e0 + j * E_CHUNK
        pltpu.sync_copy(rowp.at[pl.ds(base, E_CHUNK)], rowbuf)
        pltpu.sync_copy(ones_v, acc.at[rowbuf], add=True)
        return carry

    lax.fori_loop(0, DEG_CHUNKS, step, 0)
    plsc.subcore_barrier()
    pltpu.sync_copy(acc.at[pl.ds(r0, ROWS_PER_TILE)],
                    deg_out.at[c, pl.ds(r0, ROWS_PER_TILE)])


def _deg_call(rowp, ones_dh, zeros_dh):
    f = pl.kernel(
        _deg_body,
        out_type=jax.ShapeDtypeStruct((2, N_PAD, DH), jnp.float32),
        mesh=_mesh(),
        scratch_types=[
            pltpu.VMEM_SHARED((N_PAD, DH), jnp.float32),
            pltpu.VMEM((E_CHUNK, DH), jnp.float32),
            pltpu.VMEM((E_CHUNK,), jnp.int32),
        ],
    )
    return f(rowp, ones_dh, zeros_dh)


G = 3
A_CHUNK = 128
A_CHUNKS = E_PAD // (N_TILES * A_CHUNK)
N_GROUPS = A_CHUNKS // G
ACC_ROWS = 10112
ACC_PER_TILE = ACC_ROWS // N_TILES


def _agg_body(sup2, col3, row3, zeros_hbm, agg_out,
              acc, rows_v, colbuf, rowbuf,
              sg0, sg1, sg2, ss0, ss1, ss2):
    c = lax.axis_index("c")
    s = lax.axis_index("s")
    r0 = s * ACC_PER_TILE
    pltpu.sync_copy(zeros_hbm, acc.at[pl.ds(r0, ACC_PER_TILE)])
    plsc.subcore_barrier()
    sgs = (sg0, sg1, sg2)
    sss = (ss0, ss1, ss2)

    def group(g, carry):
        sg = s * N_GROUPS + g
        pltpu.sync_copy(col3.at[c, sg], colbuf)
        pltpu.sync_copy(row3.at[sg], rowbuf)
        gdescs = [
            pltpu.async_copy(sup2.at[colbuf.at[k]], rows_v.at[k], sgs[k])
            for k in range(G)
        ]
        sdescs = []
        for k in range(G):
            gdescs[k].wait()
            sdescs.append(pltpu.async_copy(
                rows_v.at[k], acc.at[rowbuf.at[k]], sss[k], add=True))
        for d in sdescs:
            d.wait()
        return carry

    lax.fori_loop(0, N_GROUPS, group, 0)
    plsc.subcore_barrier()
    pltpu.sync_copy(acc.at[pl.ds(r0, ACC_PER_TILE)],
                    agg_out.at[c, pl.ds(r0, ACC_PER_TILE)])


def _agg_call(sup2, col3, row3, zeros_dh):
    f = pl.kernel(
        _agg_body,
        out_type=jax.ShapeDtypeStruct((2, N_PAD, DH), jnp.float32),
        mesh=_mesh(),
        scratch_types=[
            pltpu.VMEM_SHARED((ACC_ROWS, DH), jnp.float32),
            pltpu.VMEM((G, A_CHUNK, DH), jnp.float32),
            pltpu.VMEM((G, A_CHUNK), jnp.int32),
            pltpu.VMEM((G, A_CHUNK), jnp.int32),
            pltpu.SemaphoreType.DMA,
            pltpu.SemaphoreType.DMA,
            pltpu.SemaphoreType.DMA,
            pltpu.SemaphoreType.DMA,
            pltpu.SemaphoreType.DMA,
            pltpu.SemaphoreType.DMA,
        ],
    )
    return f(sup2, col3, row3, zeros_dh)


def _dinv_block(deg_ref):
    deg = deg_ref[0, :, 0:1] + deg_ref[1, :, 0:1]
    return jnp.where(deg > 0.0, lax.rsqrt(jnp.maximum(deg, 1.0)), 0.0)


def _mm1_body(x_ref, w_ref, deg_ref, sup_ref):
    dinv = _dinv_block(deg_ref)
    sup = jnp.dot(x_ref[...], w_ref[...], preferred_element_type=jnp.float32)
    sup = sup * dinv
    sup_ref[0] = sup[:, :DH]
    sup_ref[1] = sup[:, DH:]


def _mm2_body(x_ref, agg_ref, deg_ref, w_ref, xn_ref, sup_ref):
    dinv = _dinv_block(deg_ref)
    agg = jnp.concatenate([agg_ref[0], agg_ref[1]], axis=1)
    xn = x_ref[...] + jnp.maximum(agg * dinv, 0.0)
    xn_ref[...] = xn
    sup = jnp.dot(xn, w_ref[...], preferred_element_type=jnp.float32)
    sup = sup * dinv
    sup_ref[0] = sup[:, :DH]
    sup_ref[1] = sup[:, DH:]


def _fin_body(agg_ref, deg_ref, out_ref):
    dinv = _dinv_block(deg_ref)
    out_ref[...] = jnp.concatenate([agg_ref[0], agg_ref[1]], axis=1) * dinv


_row_spec = pl.BlockSpec((BM, D), lambda i: (i, 0))
_half2_spec = pl.BlockSpec((2, BM, DH), lambda i: (0, i, 0))
_deg_spec = pl.BlockSpec((2, BM, DH), lambda i: (0, i, 0))
_w_spec = pl.BlockSpec((D, D), lambda i: (0, 0))

_half2_out = jax.ShapeDtypeStruct((2, N_PAD, DH), jnp.float32)
_row_out = jax.ShapeDtypeStruct((N_PAD, D), jnp.float32)

_mm1 = pl.pallas_call(
    _mm1_body,
    grid=(GRID,),
    in_specs=[_row_spec, _w_spec, _deg_spec],
    out_specs=_half2_spec,
    out_shape=_half2_out,
)

_mm2 = pl.pallas_call(
    _mm2_body,
    grid=(GRID,),
    in_specs=[_row_spec, _half2_spec, _deg_spec, _w_spec],
    out_specs=[_row_spec, _half2_spec],
    out_shape=[_row_out, _half2_out],
)

_fin = pl.pallas_call(
    _fin_body,
    grid=(GRID,),
    in_specs=[_half2_spec, _deg_spec],
    out_specs=_row_spec,
    out_shape=_row_out,
)


def kernel(x, edge_index, W1, W2, W3):
    x = x.astype(jnp.float32)
    xp = jnp.pad(x, ((0, N_PAD - N_NODES), (0, 0)))
    loop = jnp.arange(N_NODES, dtype=jnp.int32)
    n_fill = E_PAD - (edge_index.shape[1] + N_NODES)
    rowp = jnp.concatenate(
        [edge_index[0], loop, jnp.full((n_fill,), DUMMY_ROW, jnp.int32)])
    colp = jnp.concatenate(
        [edge_index[1], loop, jnp.zeros((n_fill,), jnp.int32)])
    n_grp = E_PAD // (G * A_CHUNK)
    col3 = jnp.stack([colp, colp + N_PAD]).reshape(2, n_grp, G, A_CHUNK)
    row3 = rowp.reshape(n_grp, G, A_CHUNK)
    ones_dh = jnp.ones((E_CHUNK, DH), jnp.float32)
    zeros_deg = jnp.zeros((ROWS_PER_TILE, DH), jnp.float32)
    zeros_dh = jnp.zeros((ACC_PER_TILE, DH), jnp.float32)

    deg = _deg_call(rowp, ones_dh, zeros_deg)
    sup = _mm1(xp, W1, deg).reshape(2 * N_PAD, DH)
    agg = _agg_call(sup, col3, row3, zeros_dh)
    x1, sup = _mm2(xp, agg, deg, W2)
    agg = _agg_call(sup.reshape(2 * N_PAD, DH), col3, row3, zeros_dh)
    _, sup = _mm2(x1, agg, deg, W3)
    agg = _agg_call(sup.reshape(2 * N_PAD, DH), col3, row3, zeros_dh)
    logits = _fin(agg, deg)
    return logits[:N_NODES]


# speedup vs baseline: 7.7457x; 1.0222x over previous
import functools

import jax
import jax.numpy as jnp
from jax import lax
from jax.experimental import pallas as pl
from jax.experimental.pallas import tpu as pltpu
from jax.experimental.pallas import tpu_sc as plsc

N_NODES = 10000
N_PAD = 10240
D = 256
DH = 128
E_CHUNK = 128
E_PAD = 172032
N_TILES = 16
ROWS_PER_TILE = N_PAD // N_TILES
AGG_CHUNKS = E_PAD // (N_TILES * E_CHUNK)
DEG_CHUNKS = E_PAD // (2 * N_TILES * E_CHUNK)
DUMMY_ROW = N_NODES + 1
BM = 256
GRID = N_PAD // BM


@functools.cache
def _mesh():
    return plsc.VectorSubcoreMesh(
        core_axis_name="c", subcore_axis_name="s", num_cores=2, num_subcores=16)


def _deg_body(rowp, ones_hbm, zeros_hbm, deg_out, acc, ones_v, rowbuf):
    c = lax.axis_index("c")
    s = lax.axis_index("s")
    r0 = s * ROWS_PER_TILE
    pltpu.sync_copy(zeros_hbm, acc.at[pl.ds(r0, ROWS_PER_TILE)])
    pltpu.sync_copy(ones_hbm, ones_v)
    plsc.subcore_barrier()
    e0 = (c * N_TILES + s) * (DEG_CHUNKS * E_CHUNK)

    def step(j, carry):
        base = e0 + j * E_CHUNK
        pltpu.sync_copy(rowp.at[pl.ds(base, E_CHUNK)], rowbuf)
        pltpu.sync_copy(ones_v, acc.at[rowbuf], add=True)
        return carry

    lax.fori_loop(0, DEG_CHUNKS, step, 0)
    plsc.subcore_barrier()
    pltpu.sync_copy(acc.at[pl.ds(r0, ROWS_PER_TILE)],
                    deg_out.at[c, pl.ds(r0, ROWS_PER_TILE)])


def _deg_call(rowp, ones_dh, zeros_dh):
    f = pl.kernel(
        _deg_body,
        out_type=jax.ShapeDtypeStruct((2, N_PAD, DH), jnp.float32),
        mesh=_mesh(),
        scratch_types=[
            pltpu.VMEM_SHARED((N_PAD, DH), jnp.float32),
            pltpu.VMEM((E_CHUNK, DH), jnp.float32),
            pltpu.VMEM((E_CHUNK,), jnp.int32),
        ],
    )
    return f(rowp, ones_dh, zeros_dh)


G = 2
A_CHUNK = 128
A_CHUNKS = E_PAD // (N_TILES * A_CHUNK)
N_GROUPS = A_CHUNKS // G
ACC_ROWS = 10112
ACC_PER_TILE = ACC_ROWS // N_TILES
IDX_BLK = 8


def _agg_body(sup2, idx5, zeros_hbm, agg_out,
              acc, rows_v, ibuf,
              sg0, sg1, ss0, ss1, si0, si1):
    c = lax.axis_index("c")
    s = lax.axis_index("s")
    r0 = s * ACC_PER_TILE
    sg_base = s * N_GROUPS
    pltpu.async_copy(idx5.at[c, sg_base], ibuf.at[pl.ds(0, IDX_BLK)], si0)
    pltpu.sync_copy(zeros_hbm, acc.at[pl.ds(r0, ACC_PER_TILE)])
    plsc.subcore_barrier()
    sgs = (sg0, sg1)
    sss = (ss0, ss1)
    sis = (si0, si1)

    def one_group(g, half):
        ib = half * IDX_BLK
        pltpu.make_async_copy(
            idx5.at[c, sg_base + g], ibuf.at[pl.ds(ib, IDX_BLK)],
            sis[half]).wait()
        gdescs = []
        for k in range(G):
            @pl.when(g > 0)
            def _(k=k):
                pltpu.make_async_copy(
                    rows_v.at[k], acc.at[pl.ds(0, A_CHUNK)], sss[k]).wait()
            gdescs.append(pltpu.async_copy(
                sup2.at[ibuf.at[ib + k]], rows_v.at[k], sgs[k]))
        @pl.when(g + 1 < N_GROUPS)
        def _():
            pltpu.async_copy(
                idx5.at[c, sg_base + g + 1],
                ibuf.at[pl.ds((1 - half) * IDX_BLK, IDX_BLK)], sis[1 - half])
        for k in range(G):
            gdescs[k].wait()
            pltpu.async_copy(
                rows_v.at[k], acc.at[ibuf.at[ib + G + k]], sss[k], add=True)

    def super_iter(gg, carry):
        one_group(2 * gg, 0)
        one_group(2 * gg + 1, 1)
        return carry

    lax.fori_loop(0, N_GROUPS // 2, super_iter, 0)
    for k in range(G):
        pltpu.make_async_copy(
            rows_v.at[k], acc.at[pl.ds(0, A_CHUNK)], sss[k]).wait()
    plsc.subcore_barrier()
    pltpu.sync_copy(acc.at[pl.ds(r0, ACC_PER_TILE)],
                    agg_out.at[c, pl.ds(r0, ACC_PER_TILE)])


def _agg_call(sup2, idx5, zeros_dh):
    f = pl.kernel(
        _agg_body,
        out_type=jax.ShapeDtypeStruct((2, N_PAD, DH), jnp.float32),
        mesh=_mesh(),
        scratch_types=[
            pltpu.VMEM_SHARED((ACC_ROWS, DH), jnp.float32),
            pltpu.VMEM((G, A_CHUNK, DH), jnp.float32),
            pltpu.VMEM((2 * IDX_BLK, A_CHUNK), jnp.int32),
            pltpu.SemaphoreType.DMA,
            pltpu.SemaphoreType.DMA,
            pltpu.SemaphoreType.DMA,
            pltpu.SemaphoreType.DMA,
            pltpu.SemaphoreType.DMA,
            pltpu.SemaphoreType.DMA,
        ],
    )
    return f(sup2, idx5, zeros_dh)


def _dinv_block(deg_ref):
    deg = deg_ref[0, :, 0:1] + deg_ref[1, :, 0:1]
    return jnp.where(deg > 0.0, lax.rsqrt(jnp.maximum(deg, 1.0)), 0.0)


def _mm1_body(x_ref, w_ref, deg_ref, sup_ref):
    dinv = _dinv_block(deg_ref)
    sup = jnp.dot(x_ref[...], w_ref[...], preferred_element_type=jnp.float32)
    sup = sup * dinv
    sup_ref[0] = sup[:, :DH]
    sup_ref[1] = sup[:, DH:]


def _mm2_body(x_ref, agg_ref, deg_ref, w_ref, xn_ref, sup_ref):
    dinv = _dinv_block(deg_ref)
    agg = jnp.concatenate([agg_ref[0], agg_ref[1]], axis=1)
    xn = x_ref[...] + jnp.maximum(agg * dinv, 0.0)
    xn_ref[...] = xn
    sup = jnp.dot(xn, w_ref[...], preferred_element_type=jnp.float32)
    sup = sup * dinv
    sup_ref[0] = sup[:, :DH]
    sup_ref[1] = sup[:, DH:]


def _fin_body(agg_ref, deg_ref, out_ref):
    dinv = _dinv_block(deg_ref)
    out_ref[...] = jnp.concatenate([agg_ref[0], agg_ref[1]], axis=1) * dinv


_row_spec = pl.BlockSpec((BM, D), lambda i: (i, 0))
_half2_spec = pl.BlockSpec((2, BM, DH), lambda i: (0, i, 0))
_deg_spec = pl.BlockSpec((2, BM, DH), lambda i: (0, i, 0))
_w_spec = pl.BlockSpec((D, D), lambda i: (0, 0))

_half2_out = jax.ShapeDtypeStruct((2, N_PAD, DH), jnp.float32)
_row_out = jax.ShapeDtypeStruct((N_PAD, D), jnp.float32)

_mm1 = pl.pallas_call(
    _mm1_body,
    grid=(GRID,),
    in_specs=[_row_spec, _w_spec, _deg_spec],
    out_specs=_half2_spec,
    out_shape=_half2_out,
)

_mm2 = pl.pallas_call(
    _mm2_body,
    grid=(GRID,),
    in_specs=[_row_spec, _half2_spec, _deg_spec, _w_spec],
    out_specs=[_row_spec, _half2_spec],
    out_shape=[_row_out, _half2_out],
)

_fin = pl.pallas_call(
    _fin_body,
    grid=(GRID,),
    in_specs=[_half2_spec, _deg_spec],
    out_specs=_row_spec,
    out_shape=_row_out,
)


def kernel(x, edge_index, W1, W2, W3):
    x = x.astype(jnp.float32)
    xp = jnp.pad(x, ((0, N_PAD - N_NODES), (0, 0)))
    loop = jnp.arange(N_NODES, dtype=jnp.int32)
    n_fill = E_PAD - (edge_index.shape[1] + N_NODES)
    rowp = jnp.concatenate(
        [edge_index[0], loop, jnp.full((n_fill,), DUMMY_ROW, jnp.int32)])
    colp = jnp.concatenate(
        [edge_index[1], loop, jnp.zeros((n_fill,), jnp.int32)])
    n_grp = E_PAD // (G * A_CHUNK)
    col3 = jnp.stack([colp, colp + N_PAD]).reshape(2, n_grp, G, A_CHUNK)
    row3 = jnp.broadcast_to(
        rowp.reshape(n_grp, G, A_CHUNK), (2, n_grp, G, A_CHUNK))
    zpad = jnp.zeros((2, n_grp, IDX_BLK - 2 * G, A_CHUNK), jnp.int32)
    idx5 = jnp.concatenate([col3, row3, zpad], axis=2)
    ones_dh = jnp.ones((E_CHUNK, DH), jnp.float32)
    zeros_deg = jnp.zeros((ROWS_PER_TILE, DH), jnp.float32)
    zeros_dh = jnp.zeros((ACC_PER_TILE, DH), jnp.float32)

    deg = _deg_call(rowp, ones_dh, zeros_deg)
    sup = _mm1(xp, W1, deg).reshape(2 * N_PAD, DH)
    agg = _agg_call(sup, idx5, zeros_dh)
    x1, sup = _mm2(xp, agg, deg, W2)
    agg = _agg_call(sup.reshape(2 * N_PAD, DH), idx5, zeros_dh)
    _, sup = _mm2(x1, agg, deg, W3)
    agg = _agg_call(sup.reshape(2 * N_PAD, DH), idx5, zeros_dh)
    logits = _fin(agg, deg)
    return logits[:N_NODES]


# speedup vs baseline: 7.9071x; 1.0208x over previous
import functools

import jax
import jax.numpy as jnp
from jax import lax
from jax.experimental import pallas as pl
from jax.experimental.pallas import tpu as pltpu
from jax.experimental.pallas import tpu_sc as plsc

N_NODES = 10000
N_PAD = 10240
D = 256
DH = 128
E_CHUNK = 128
E_PAD = 172032
N_TILES = 16
ROWS_PER_TILE = N_PAD // N_TILES
AGG_CHUNKS = E_PAD // (N_TILES * E_CHUNK)
DEG_CHUNKS = E_PAD // (2 * N_TILES * E_CHUNK)
DUMMY_ROW = N_NODES + 1
BM = 256
GRID = N_PAD // BM


@functools.cache
def _mesh():
    return plsc.VectorSubcoreMesh(
        core_axis_name="c", subcore_axis_name="s", num_cores=2, num_subcores=16)


DEG_GRP = 7
DEG_NGRP = DEG_CHUNKS // DEG_GRP


def _deg_body(rowg, ones_hbm, zeros_hbm, deg_out, acc, ones_v, ibuf):
    c = lax.axis_index("c")
    s = lax.axis_index("s")
    r0 = s * ROWS_PER_TILE
    g_base = (c * N_TILES + s) * DEG_NGRP

    def scoped(si0, si1, ss0, ss1):
        sis = (si0, si1)
        sss = (ss0, ss1)
        pltpu.async_copy(rowg.at[g_base], ibuf.at[pl.ds(0, 8)], si0)
        pltpu.sync_copy(zeros_hbm, acc.at[pl.ds(r0, ROWS_PER_TILE)])
        pltpu.sync_copy(ones_hbm, ones_v)
        plsc.subcore_barrier()
        for g in range(DEG_NGRP):
            h = g % 2
            pltpu.make_async_copy(
                rowg.at[g_base + g], ibuf.at[pl.ds(h * 8, 8)], sis[h]).wait()
            if g >= 1:
                for _ in range(DEG_GRP):
                    pltpu.make_async_copy(
                        ones_v, acc.at[pl.ds(0, E_CHUNK)], sss[1 - h]).wait()
            if g + 1 < DEG_NGRP:
                pltpu.async_copy(
                    rowg.at[g_base + g + 1],
                    ibuf.at[pl.ds((1 - h) * 8, 8)], sis[1 - h])
            for r in range(DEG_GRP):
                pltpu.async_copy(
                    ones_v, acc.at[ibuf.at[h * 8 + r]], sss[h], add=True)
        for _ in range(DEG_GRP):
            pltpu.make_async_copy(
                ones_v, acc.at[pl.ds(0, E_CHUNK)],
                sss[(DEG_NGRP - 1) % 2]).wait()
        plsc.subcore_barrier()
        pltpu.sync_copy(acc.at[pl.ds(r0, ROWS_PER_TILE)],
                        deg_out.at[c, pl.ds(r0, ROWS_PER_TILE)])

    pl.run_scoped(scoped,
                  pltpu.SemaphoreType.DMA, pltpu.SemaphoreType.DMA,
                  pltpu.SemaphoreType.DMA, pltpu.SemaphoreType.DMA)


def _deg_call(rowg, ones_dh, zeros_dh):
    f = pl.kernel(
        _deg_body,
        out_type=jax.ShapeDtypeStruct((2, N_PAD, DH), jnp.float32),
        mesh=_mesh(),
        scratch_types=[
            pltpu.VMEM_SHARED((N_PAD, DH), jnp.float32),
            pltpu.VMEM((E_CHUNK, DH), jnp.float32),
            pltpu.VMEM((16, E_CHUNK), jnp.int32),
        ],
    )
    return f(rowg, ones_dh, zeros_dh)


G = 2
A_CHUNK = 128
A_CHUNKS = E_PAD // (N_TILES * A_CHUNK)
N_GROUPS = A_CHUNKS // G
ACC_ROWS = 10112
ACC_PER_TILE = ACC_ROWS // N_TILES
IDX_BLK = 8


def _agg_body(sup2, idx5, zeros_hbm, agg_out,
              acc, rows_v, ibuf,
              sg0, sg1, ss0, ss1, si0, si1):
    c = lax.axis_index("c")
    s = lax.axis_index("s")
    r0 = s * ACC_PER_TILE
    sg_base = s * N_GROUPS
    pltpu.async_copy(idx5.at[c, sg_base], ibuf.at[pl.ds(0, IDX_BLK)], si0)
    pltpu.sync_copy(zeros_hbm, acc.at[pl.ds(r0, ACC_PER_TILE)])
    plsc.subcore_barrier()
    sgs = (sg0, sg1)
    sss = (ss0, ss1)
    sis = (si0, si1)

    def one_group(g, half):
        ib = half * IDX_BLK
        pltpu.make_async_copy(
            idx5.at[c, sg_base + g], ibuf.at[pl.ds(ib, IDX_BLK)],
            sis[half]).wait()
        gdescs = []
        for k in range(G):
            @pl.when(g > 0)
            def _(k=k):
                pltpu.make_async_copy(
                    rows_v.at[k], acc.at[pl.ds(0, A_CHUNK)], sss[k]).wait()
            gdescs.append(pltpu.async_copy(
                sup2.at[ibuf.at[ib + k]], rows_v.at[k], sgs[k]))
        @pl.when(g + 1 < N_GROUPS)
        def _():
            pltpu.async_copy(
                idx5.at[c, sg_base + g + 1],
                ibuf.at[pl.ds((1 - half) * IDX_BLK, IDX_BLK)], sis[1 - half])
        for k in range(G):
            gdescs[k].wait()
            pltpu.async_copy(
                rows_v.at[k], acc.at[ibuf.at[ib + G + k]], sss[k], add=True)

    def super_iter(gg, carry):
        one_group(2 * gg, 0)
        one_group(2 * gg + 1, 1)
        return carry

    lax.fori_loop(0, N_GROUPS // 2, super_iter, 0)
    for k in range(G):
        pltpu.make_async_copy(
            rows_v.at[k], acc.at[pl.ds(0, A_CHUNK)], sss[k]).wait()
    plsc.subcore_barrier()
    pltpu.sync_copy(acc.at[pl.ds(r0, ACC_PER_TILE)],
                    agg_out.at[c, pl.ds(r0, ACC_PER_TILE)])


def _agg_call(sup2, idx5, zeros_dh):
    f = pl.kernel(
        _agg_body,
        out_type=jax.ShapeDtypeStruct((2, N_PAD, DH), jnp.float32),
        mesh=_mesh(),
        scratch_types=[
            pltpu.VMEM_SHARED((ACC_ROWS, DH), jnp.float32),
            pltpu.VMEM((G, A_CHUNK, DH), jnp.float32),
            pltpu.VMEM((2 * IDX_BLK, A_CHUNK), jnp.int32),
            pltpu.SemaphoreType.DMA,
            pltpu.SemaphoreType.DMA,
            pltpu.SemaphoreType.DMA,
            pltpu.SemaphoreType.DMA,
            pltpu.SemaphoreType.DMA,
            pltpu.SemaphoreType.DMA,
        ],
    )
    return f(sup2, idx5, zeros_dh)


def _dinv_block(deg_ref):
    deg = deg_ref[0, :, 0:1] + deg_ref[1, :, 0:1]
    return jnp.where(deg > 0.0, lax.rsqrt(jnp.maximum(deg, 1.0)), 0.0)


def _mm1_body(x_ref, w_ref, deg_ref, sup_ref):
    dinv = _dinv_block(deg_ref)
    sup = jnp.dot(x_ref[...], w_ref[...], preferred_element_type=jnp.float32)
    sup = sup * dinv
    sup_ref[0] = sup[:, :DH]
    sup_ref[1] = sup[:, DH:]


def _mm2_body(x_ref, agg_ref, deg_ref, w_ref, xn_ref, sup_ref):
    dinv = _dinv_block(deg_ref)
    agg = jnp.concatenate([agg_ref[0], agg_ref[1]], axis=1)
    xn = x_ref[...] + jnp.maximum(agg * dinv, 0.0)
    xn_ref[...] = xn
    sup = jnp.dot(xn, w_ref[...], preferred_element_type=jnp.float32)
    sup = sup * dinv
    sup_ref[0] = sup[:, :DH]
    sup_ref[1] = sup[:, DH:]


def _mm3_body(x_ref, agg_ref, deg_ref, w_ref, sup_ref):
    dinv = _dinv_block(deg_ref)
    agg = jnp.concatenate([agg_ref[0], agg_ref[1]], axis=1)
    xn = x_ref[...] + jnp.maximum(agg * dinv, 0.0)
    sup = jnp.dot(xn, w_ref[...], preferred_element_type=jnp.float32)
    sup = sup * dinv
    sup_ref[0] = sup[:, :DH]
    sup_ref[1] = sup[:, DH:]


def _fin_body(agg_ref, deg_ref, out_ref):
    dinv = _dinv_block(deg_ref)
    out_ref[...] = jnp.concatenate([agg_ref[0], agg_ref[1]], axis=1) * dinv


_row_spec = pl.BlockSpec((BM, D), lambda i: (i, 0))
_half2_spec = pl.BlockSpec((2, BM, DH), lambda i: (0, i, 0))
_deg_spec = pl.BlockSpec((2, BM, DH), lambda i: (0, i, 0))
_w_spec = pl.BlockSpec((D, D), lambda i: (0, 0))

_half2_out = jax.ShapeDtypeStruct((2, N_PAD, DH), jnp.float32)
_row_out = jax.ShapeDtypeStruct((N_PAD, D), jnp.float32)

_mm1 = pl.pallas_call(
    _mm1_body,
    grid=(GRID,),
    in_specs=[_row_spec, _w_spec, _deg_spec],
    out_specs=_half2_spec,
    out_shape=_half2_out,
)

_mm2 = pl.pallas_call(
    _mm2_body,
    grid=(GRID,),
    in_specs=[_row_spec, _half2_spec, _deg_spec, _w_spec],
    out_specs=[_row_spec, _half2_spec],
    out_shape=[_row_out, _half2_out],
)

_mm3 = pl.pallas_call(
    _mm3_body,
    grid=(GRID,),
    in_specs=[_row_spec, _half2_spec, _deg_spec, _w_spec],
    out_specs=_half2_spec,
    out_shape=_half2_out,
)

_fin = pl.pallas_call(
    _fin_body,
    grid=(GRID,),
    in_specs=[_half2_spec, _deg_spec],
    out_specs=_row_spec,
    out_shape=_row_out,
)


def kernel(x, edge_index, W1, W2, W3):
    x = x.astype(jnp.float32)
    xp = jnp.pad(x, ((0, N_PAD - N_NODES), (0, 0)))
    loop = jnp.arange(N_NODES, dtype=jnp.int32)
    n_fill = E_PAD - (edge_index.shape[1] + N_NODES)
    rowp = jnp.concatenate(
        [edge_index[0], loop, jnp.full((n_fill,), DUMMY_ROW, jnp.int32)])
    colp = jnp.concatenate(
        [edge_index[1], loop, jnp.zeros((n_fill,), jnp.int32)])
    n_grp = E_PAD // (G * A_CHUNK)
    col3 = jnp.stack([colp, colp + N_PAD]).reshape(2, n_grp, G, A_CHUNK)
    row3 = jnp.broadcast_to(
        rowp.reshape(n_grp, G, A_CHUNK), (2, n_grp, G, A_CHUNK))
    zpad = jnp.zeros((2, n_grp, IDX_BLK - 2 * G, A_CHUNK), jnp.int32)
    idx5 = jnp.concatenate([col3, row3, zpad], axis=2)
    ones_dh = jnp.ones((E_CHUNK, DH), jnp.float32)
    zeros_deg = jnp.zeros((ROWS_PER_TILE, DH), jnp.float32)
    zeros_dh = jnp.zeros((ACC_PER_TILE, DH), jnp.float32)

    rowg = jnp.pad(rowp.reshape(2 * N_TILES * DEG_NGRP, DEG_GRP, E_CHUNK),
                   ((0, 0), (0, 8 - DEG_GRP), (0, 0)))
    deg = _deg_call(rowg, ones_dh, zeros_deg)
    sup = _mm1(xp, W1, deg).reshape(2 * N_PAD, DH)
    agg = _agg_call(sup, idx5, zeros_dh)
    x1, sup = _mm2(xp, agg, deg, W2)
    agg = _agg_call(sup.reshape(2 * N_PAD, DH), idx5, zeros_dh)
    sup = _mm3(x1, agg, deg, W3)
    agg = _agg_call(sup.reshape(2 * N_PAD, DH), idx5, zeros_dh)
    logits = _fin(agg, deg)
    return logits[:N_NODES]


# speedup vs baseline: 7.9086x; 1.0002x over previous
import functools

import jax
import jax.numpy as jnp
from jax import lax
from jax.experimental import pallas as pl
from jax.experimental.pallas import tpu as pltpu
from jax.experimental.pallas import tpu_sc as plsc

N_NODES = 10000
N_PAD = 10240
D = 256
DH = 128
E_CHUNK = 128
E_PAD = 172032
N_TILES = 16
ROWS_PER_TILE = N_PAD // N_TILES
AGG_CHUNKS = E_PAD // (N_TILES * E_CHUNK)
DEG_CHUNKS = E_PAD // (2 * N_TILES * E_CHUNK)
DUMMY_ROW = N_NODES + 1
BM = 256
GRID = N_PAD // BM


@functools.cache
def _mesh():
    return plsc.VectorSubcoreMesh(
        core_axis_name="c", subcore_axis_name="s", num_cores=2, num_subcores=16)


DEG_GRP = 7
DEG_NGRP = DEG_CHUNKS // DEG_GRP


def _deg_body(rowg, ones_hbm, zeros_hbm, deg_out, acc, ones_v, ibuf):
    c = lax.axis_index("c")
    s = lax.axis_index("s")
    r0 = s * ROWS_PER_TILE
    g_base = (c * N_TILES + s) * DEG_NGRP

    def scoped(si0, si1, ss0, ss1):
        sis = (si0, si1)
        sss = (ss0, ss1)
        pltpu.async_copy(rowg.at[g_base], ibuf.at[pl.ds(0, 8)], si0)
        pltpu.sync_copy(zeros_hbm, acc.at[pl.ds(r0, ROWS_PER_TILE)])
        pltpu.sync_copy(ones_hbm, ones_v)
        plsc.subcore_barrier()
        for g in range(DEG_NGRP):
            h = g % 2
            pltpu.make_async_copy(
                rowg.at[g_base + g], ibuf.at[pl.ds(h * 8, 8)], sis[h]).wait()
            if g >= 1:
                for _ in range(DEG_GRP):
                    pltpu.make_async_copy(
                        ones_v, acc.at[pl.ds(0, E_CHUNK)], sss[1 - h]).wait()
            if g + 1 < DEG_NGRP:
                pltpu.async_copy(
                    rowg.at[g_base + g + 1],
                    ibuf.at[pl.ds((1 - h) * 8, 8)], sis[1 - h])
            for r in range(DEG_GRP):
                pltpu.async_copy(
                    ones_v, acc.at[ibuf.at[h * 8 + r]], sss[h], add=True)
        for _ in range(DEG_GRP):
            pltpu.make_async_copy(
                ones_v, acc.at[pl.ds(0, E_CHUNK)],
                sss[(DEG_NGRP - 1) % 2]).wait()
        plsc.subcore_barrier()
        pltpu.sync_copy(acc.at[pl.ds(r0, ROWS_PER_TILE)],
                        deg_out.at[c, pl.ds(r0, ROWS_PER_TILE)])

    pl.run_scoped(scoped,
                  pltpu.SemaphoreType.DMA, pltpu.SemaphoreType.DMA,
                  pltpu.SemaphoreType.DMA, pltpu.SemaphoreType.DMA)


def _deg_call(rowg, ones_dh, zeros_dh):
    f = pl.kernel(
        _deg_body,
        out_type=jax.ShapeDtypeStruct((2, N_PAD, DH), jnp.float32),
        mesh=_mesh(),
        scratch_types=[
            pltpu.VMEM_SHARED((N_PAD, DH), jnp.float32),
            pltpu.VMEM((E_CHUNK, DH), jnp.float32),
            pltpu.VMEM((16, E_CHUNK), jnp.int32),
        ],
    )
    return f(rowg, ones_dh, zeros_dh)


G = 2
A_CHUNK = 128
A_CHUNKS = E_PAD // (N_TILES * A_CHUNK)
N_GROUPS = A_CHUNKS // G
ACC_ROWS = 10112
ACC_PER_TILE = ACC_ROWS // N_TILES
IDX_BLK = 8


def _agg_body(sup2, idx5, zeros_hbm, agg_out,
              acc, rows_v, ibuf,
              sg0, sg1, ss0, ss1, si0, si1):
    c = lax.axis_index("c")
    s = lax.axis_index("s")
    r0 = s * ACC_PER_TILE
    sg_base = s * N_GROUPS
    pltpu.async_copy(idx5.at[c, sg_base], ibuf.at[pl.ds(0, IDX_BLK)], si0)
    pltpu.sync_copy(zeros_hbm, acc.at[pl.ds(r0, ACC_PER_TILE)])
    plsc.subcore_barrier()
    sgs = (sg0, sg1)
    sss = (ss0, ss1)
    sis = (si0, si1)

    def one_group(g, half):
        ib = half * IDX_BLK
        pltpu.make_async_copy(
            idx5.at[c, sg_base + g], ibuf.at[pl.ds(ib, IDX_BLK)],
            sis[half]).wait()
        gdescs = []
        for k in range(G):
            @pl.when(g > 0)
            def _(k=k):
                pltpu.make_async_copy(
                    rows_v.at[k], acc.at[pl.ds(0, A_CHUNK)], sss[k]).wait()
            gdescs.append(pltpu.async_copy(
                sup2.at[ibuf.at[ib + k]], rows_v.at[k], sgs[k]))
        @pl.when(g + 1 < N_GROUPS)
        def _():
            pltpu.async_copy(
                idx5.at[c, sg_base + g + 1],
                ibuf.at[pl.ds((1 - half) * IDX_BLK, IDX_BLK)], sis[1 - half])
        for k in range(G):
            gdescs[k].wait()
            pltpu.async_copy(
                rows_v.at[k], acc.at[ibuf.at[ib + G + k]], sss[k], add=True)

    def super_iter(gg, carry):
        one_group(2 * gg, 0)
        one_group(2 * gg + 1, 1)
        return carry

    lax.fori_loop(0, N_GROUPS // 2, super_iter, 0)
    for k in range(G):
        pltpu.make_async_copy(
            rows_v.at[k], acc.at[pl.ds(0, A_CHUNK)], sss[k]).wait()
    plsc.subcore_barrier()
    pltpu.sync_copy(acc.at[pl.ds(r0, ACC_PER_TILE)],
                    agg_out.at[c, pl.ds(r0, ACC_PER_TILE)])


def _agg_call(sup2, idx5, zeros_dh):
    f = pl.kernel(
        _agg_body,
        out_type=jax.ShapeDtypeStruct((2, N_PAD, DH), jnp.float32),
        mesh=_mesh(),
        scratch_types=[
            pltpu.VMEM_SHARED((ACC_ROWS, DH), jnp.float32),
            pltpu.VMEM((G, A_CHUNK, DH), jnp.float32),
            pltpu.VMEM((2 * IDX_BLK, A_CHUNK), jnp.int32),
            pltpu.SemaphoreType.DMA,
            pltpu.SemaphoreType.DMA,
            pltpu.SemaphoreType.DMA,
            pltpu.SemaphoreType.DMA,
            pltpu.SemaphoreType.DMA,
            pltpu.SemaphoreType.DMA,
        ],
    )
    return f(sup2, idx5, zeros_dh)


def _dinv_block(deg_ref):
    deg = deg_ref[0, :, 0:1] + deg_ref[1, :, 0:1]
    return jnp.where(deg > 0.0, lax.rsqrt(jnp.maximum(deg, 1.0)), 0.0)


def _mm1_body(x_ref, w_ref, deg_ref, sup_ref):
    dinv = _dinv_block(deg_ref)
    sup = jnp.dot(x_ref[...], w_ref[...], preferred_element_type=jnp.float32)
    sup = sup * dinv
    sup_ref[0] = sup[:, :DH]
    sup_ref[1] = sup[:, DH:]


def _mm2_body(x_ref, agg_ref, deg_ref, w_ref, xn_ref, sup_ref):
    dinv = _dinv_block(deg_ref)
    agg = jnp.concatenate([agg_ref[0], agg_ref[1]], axis=1)
    xn = x_ref[...] + jnp.maximum(agg * dinv, 0.0)
    xn_ref[...] = xn
    sup = jnp.dot(xn, w_ref[...], preferred_element_type=jnp.float32)
    sup = sup * dinv
    sup_ref[0] = sup[:, :DH]
    sup_ref[1] = sup[:, DH:]


def _mm3_body(x_ref, agg_ref, deg_ref, w_ref, sup_ref):
    dinv = _dinv_block(deg_ref)
    agg = jnp.concatenate([agg_ref[0], agg_ref[1]], axis=1)
    xn = x_ref[...] + jnp.maximum(agg * dinv, 0.0)
    sup = jnp.dot(xn, w_ref[...], preferred_element_type=jnp.float32)
    sup = sup * dinv
    sup_ref[0] = sup[:, :DH]
    sup_ref[1] = sup[:, DH:]


def _fin_body(agg_ref, deg_ref, out_ref):
    dinv = _dinv_block(deg_ref)
    out_ref[...] = jnp.concatenate([agg_ref[0], agg_ref[1]], axis=1) * dinv


_row_spec = pl.BlockSpec((BM, D), lambda i: (i, 0))
_half2_spec = pl.BlockSpec((2, BM, DH), lambda i: (0, i, 0))
_deg_spec = pl.BlockSpec((2, BM, 1), lambda i: (0, i, 0))
_w_spec = pl.BlockSpec((D, D), lambda i: (0, 0))

_half2_out = jax.ShapeDtypeStruct((2, N_PAD, DH), jnp.float32)
_row_out = jax.ShapeDtypeStruct((N_PAD, D), jnp.float32)

_mm1 = pl.pallas_call(
    _mm1_body,
    grid=(GRID,),
    in_specs=[_row_spec, _w_spec, _deg_spec],
    out_specs=_half2_spec,
    out_shape=_half2_out,
)

_mm2 = pl.pallas_call(
    _mm2_body,
    grid=(GRID,),
    in_specs=[_row_spec, _half2_spec, _deg_spec, _w_spec],
    out_specs=[_row_spec, _half2_spec],
    out_shape=[_row_out, _half2_out],
)

_mm3 = pl.pallas_call(
    _mm3_body,
    grid=(GRID,),
    in_specs=[_row_spec, _half2_spec, _deg_spec, _w_spec],
    out_specs=_half2_spec,
    out_shape=_half2_out,
)

_fin = pl.pallas_call(
    _fin_body,
    grid=(GRID,),
    in_specs=[_half2_spec, _deg_spec],
    out_specs=_row_spec,
    out_shape=_row_out,
)


def kernel(x, edge_index, W1, W2, W3):
    x = x.astype(jnp.float32)
    xp = jnp.pad(x, ((0, N_PAD - N_NODES), (0, 0)))
    loop = jnp.arange(N_NODES, dtype=jnp.int32)
    n_fill = E_PAD - (edge_index.shape[1] + N_NODES)
    rowp = jnp.concatenate(
        [edge_index[0], loop, jnp.full((n_fill,), DUMMY_ROW, jnp.int32)])
    colp = jnp.concatenate(
        [edge_index[1], loop, jnp.zeros((n_fill,), jnp.int32)])
    n_grp = E_PAD // (G * A_CHUNK)
    col3 = jnp.stack([colp, colp + N_PAD]).reshape(2, n_grp, G, A_CHUNK)
    row3 = jnp.broadcast_to(
        rowp.reshape(n_grp, G, A_CHUNK), (2, n_grp, G, A_CHUNK))
    zpad = jnp.zeros((2, n_grp, IDX_BLK - 2 * G, A_CHUNK), jnp.int32)
    idx5 = jnp.concatenate([col3, row3, zpad], axis=2)
    ones_dh = jnp.ones((E_CHUNK, DH), jnp.float32)
    zeros_deg = jnp.zeros((ROWS_PER_TILE, DH), jnp.float32)
    zeros_dh = jnp.zeros((ACC_PER_TILE, DH), jnp.float32)

    rowg = jnp.pad(rowp.reshape(2 * N_TILES * DEG_NGRP, DEG_GRP, E_CHUNK),
                   ((0, 0), (0, 8 - DEG_GRP), (0, 0)))
    deg = _deg_call(rowg, ones_dh, zeros_deg)[:, :, :1]
    sup = _mm1(xp, W1, deg).reshape(2 * N_PAD, DH)
    agg = _agg_call(sup, idx5, zeros_dh)
    x1, sup = _mm2(xp, agg, deg, W2)
    agg = _agg_call(sup.reshape(2 * N_PAD, DH), idx5, zeros_dh)
    sup = _mm3(x1, agg, deg, W3)
    agg = _agg_call(sup.reshape(2 * N_PAD, DH), idx5, zeros_dh)
    logits = _fin(agg, deg)
    return logits[:N_NODES]
